# baseline (speedup 1.0000x reference)
import jax
import jax.numpy as jnp
from jax.experimental import pallas as pl
from jax.experimental.pallas import tpu as pltpu

N = 10000
E = 320000
D = 128
DO = 64
B = 32
H = 8
HD = D // H
S1 = 75
EC = 1000
EG = E // EC
NEG = -1e9
SCALE = 1.0 / (128.0 ** 0.5)


def _scatter1_kernel(eb_ref, x_ref, agg_ref, deg_ref):
    @pl.when(pl.program_id(0) == 0)
    def _():
        agg_ref[...] = jnp.zeros_like(agg_ref)
        deg_ref[...] = jnp.zeros_like(deg_ref)

    def body(j, carry):
        s = eb_ref[0, 0, j]
        d = eb_ref[0, 1, j]
        row = x_ref[pl.ds(s, 1), :]
        agg_ref[pl.ds(d, 1), :] = agg_ref[pl.ds(d, 1), :] + row
        deg_ref[pl.ds(d, 1), :] = deg_ref[pl.ds(d, 1), :] + 1.0
        return carry

    jax.lax.fori_loop(0, EC, body, 0)


def _bn(h, g, b):
    m = jnp.mean(h, axis=0, keepdims=True)
    v = jnp.mean(jnp.square(h - m), axis=0, keepdims=True)
    return (h - m) * jax.lax.rsqrt(v + 1e-5) * g + b


def _ln(h, g, b):
    m = jnp.mean(h, axis=-1, keepdims=True)
    v = jnp.mean(jnp.square(h - m), axis=-1, keepdims=True)
    return (h - m) * jax.lax.rsqrt(v + 1e-5) * g + b


def _dense_kernel(x_ref, agg_ref, deg_ref,
                  w1_ref, b1_ref, g1_ref, bb1_ref,
                  w2_ref, b2_ref, g2_ref, bb2_ref,
                  lw_ref, lb_ref, kw_ref, vw_ref,
                  hwk_ref, hwv_ref, dinv_ref):
    h = x_ref[...] + agg_ref[...]
    h = jnp.maximum(_bn(jnp.dot(h, w1_ref[...], preferred_element_type=jnp.float32)
                        + b1_ref[...], g1_ref[...], bb1_ref[...]), 0.0)
    h = jnp.maximum(_bn(jnp.dot(h, w2_ref[...], preferred_element_type=jnp.float32)
                        + b2_ref[...], g2_ref[...], bb2_ref[...]), 0.0)
    h1 = jnp.dot(h, lw_ref[...], preferred_element_type=jnp.float32) + lb_ref[...]
    hwk_ref[...] = jnp.dot(h1, kw_ref[...], preferred_element_type=jnp.float32)
    hwv_ref[...] = jnp.dot(h1, vw_ref[...], preferred_element_type=jnp.float32)
    dinv_ref[...] = jax.lax.rsqrt(deg_ref[...] + 1.0)


def _scatter2_kernel(eb_ref, hwk_ref, hwv_ref, dinv_ref, kb_ref, vb_ref,
                     kout_ref, vout_ref):
    @pl.when(pl.program_id(0) == 0)
    def _():
        d2 = dinv_ref[...] * dinv_ref[...]
        kout_ref[...] = hwk_ref[...] * d2 + kb_ref[...]
        vout_ref[...] = hwv_ref[...] * d2 + vb_ref[...]

    def body(j, carry):
        s = eb_ref[0, 0, j]
        d = eb_ref[0, 1, j]
        coef = dinv_ref[pl.ds(s, 1), :] * dinv_ref[pl.ds(d, 1), :]
        rk = hwk_ref[pl.ds(s, 1), :] * coef
        rv = hwv_ref[pl.ds(s, 1), :] * coef
        kout_ref[pl.ds(d, 1), :] = kout_ref[pl.ds(d, 1), :] + rk
        vout_ref[pl.ds(d, 1), :] = vout_ref[pl.ds(d, 1), :] + rv
        return carry

    jax.lax.fori_loop(0, EC, body, 0)


def _mab_heads(qp, k, v, mask_row):
    outs = []
    for h in range(H):
        qh = qp[:, h * HD:(h + 1) * HD]
        kh = k[:, h * HD:(h + 1) * HD]
        vh = v[:, h * HD:(h + 1) * HD]
        s = jax.lax.dot_general(qh, kh, (((1,), (1,)), ((), ())),
                                preferred_element_type=jnp.float32) * SCALE
        if mask_row is not None:
            s = s + mask_row
        m = jnp.max(s, axis=-1, keepdims=True)
        e = jnp.exp(s - m)
        a = e / jnp.sum(e, axis=-1, keepdims=True)
        outs.append(qh + jnp.dot(a, vh, preferred_element_type=jnp.float32))
    return jnp.concatenate(outs, axis=1)


def _mab_tail(o, ow, ob, g0, b0, g1, b1):
    o = _ln(o, g0, b0)
    o = o + jnp.maximum(jnp.dot(o, ow, preferred_element_type=jnp.float32) + ob, 0.0)
    return _ln(o, g1, b1)


def _readout_kernel(kn_ref, vn_ref, brow_ref, s1_ref,
                    q1w_ref, q1b_ref, o1w_ref, o1b_ref,
                    l10g_ref, l10b_ref, l11g_ref, l11b_ref,
                    q2w_ref, q2b_ref, k2w_ref, k2b_ref, v2w_ref, v2b_ref,
                    o2w_ref, o2b_ref, l20g_ref, l20b_ref, l21g_ref, l21b_ref,
                    s3_ref, q3w_ref, q3b_ref, k3w_ref, k3b_ref, v3w_ref, v3b_ref,
                    o3w_ref, o3b_ref, l30g_ref, l30b_ref, l31g_ref, l31b_ref,
                    l2w_ref, l2b_ref, out_ref):
    b = pl.program_id(0)
    mask_row = jnp.where(brow_ref[...] == b, 0.0, NEG)
    qp1 = jnp.dot(s1_ref[...], q1w_ref[...],
                  preferred_element_type=jnp.float32) + q1b_ref[...]
    o = _mab_heads(qp1, kn_ref[...], vn_ref[...], mask_row)
    out1 = _mab_tail(o, o1w_ref[...], o1b_ref[...], l10g_ref[...], l10b_ref[...],
                     l11g_ref[...], l11b_ref[...])
    qp2 = jnp.dot(out1, q2w_ref[...], preferred_element_type=jnp.float32) + q2b_ref[...]
    k2 = jnp.dot(out1, k2w_ref[...], preferred_element_type=jnp.float32) + k2b_ref[...]
    v2 = jnp.dot(out1, v2w_ref[...], preferred_element_type=jnp.float32) + v2b_ref[...]
    o = _mab_heads(qp2, k2, v2, None)
    out2 = _mab_tail(o, o2w_ref[...], o2b_ref[...], l20g_ref[...], l20b_ref[...],
                     l21g_ref[...], l21b_ref[...])
    qp3 = jnp.dot(s3_ref[...], q3w_ref[...],
                  preferred_element_type=jnp.float32) + q3b_ref[...]
    k3 = jnp.dot(out2, k3w_ref[...], preferred_element_type=jnp.float32) + k3b_ref[...]
    v3 = jnp.dot(out2, v3w_ref[...], preferred_element_type=jnp.float32) + v3b_ref[...]
    o = _mab_heads(qp3, k3, v3, None)
    out3 = _mab_tail(o, o3w_ref[...], o3b_ref[...], l30g_ref[...], l30b_ref[...],
                     l31g_ref[...], l31b_ref[...])
    y = jnp.dot(out3, l2w_ref[...],
                preferred_element_type=jnp.float32) + l2b_ref[...]
    out_ref[...] = y.reshape(1, 1, DO)


def _full(shape):
    return pl.BlockSpec(shape, lambda i: tuple(0 for _ in shape))


def kernel(x, edge_index, batch, params):
    p = params
    ei = edge_index.astype(jnp.int32).reshape(2, EG, EC).transpose(1, 0, 2)
    brow = batch.astype(jnp.int32).reshape(1, N)
    r = lambda a: a.reshape(1, -1)

    agg, deg = pl.pallas_call(
        _scatter1_kernel,
        grid=(EG,),
        in_specs=[pl.BlockSpec((1, 2, EC), lambda i: (i, 0, 0), memory_space=pltpu.SMEM),
                  _full((N, D))],
        out_specs=[_full((N, D)), _full((N, 1))],
        out_shape=[jax.ShapeDtypeStruct((N, D), jnp.float32),
                   jax.ShapeDtypeStruct((N, 1), jnp.float32)],
    )(ei, x)

    hwk, hwv, dinv = pl.pallas_call(
        _dense_kernel,
        out_shape=[jax.ShapeDtypeStruct((N, D), jnp.float32),
                   jax.ShapeDtypeStruct((N, D), jnp.float32),
                   jax.ShapeDtypeStruct((N, 1), jnp.float32)],
    )(x, agg, deg,
      p['gin1_W'], r(p['gin1_b']), r(p['gin_bn1_g']), r(p['gin_bn1_b']),
      p['gin2_W'], r(p['gin2_b']), r(p['gin_bn2_g']), r(p['gin_bn2_b']),
      p['lin1_W'], r(p['lin1_b']), p['p1_k_W'], p['p1_v_W'])

    kn, vn = pl.pallas_call(
        _scatter2_kernel,
        grid=(EG,),
        in_specs=[pl.BlockSpec((1, 2, EC), lambda i: (i, 0, 0), memory_space=pltpu.SMEM),
                  _full((N, D)), _full((N, D)), _full((N, 1)),
                  _full((1, D)), _full((1, D))],
        out_specs=[_full((N, D)), _full((N, D))],
        out_shape=[jax.ShapeDtypeStruct((N, D), jnp.float32),
                   jax.ShapeDtypeStruct((N, D), jnp.float32)],
    )(ei, hwk, hwv, dinv, r(p['p1_k_b']), r(p['p1_v_b']))

    ro_in = [kn, vn, brow, p['p1_S'].reshape(S1, D),
             p['p1_q_W'], r(p['p1_q_b']), p['p1_o_W'], r(p['p1_o_b']),
             r(p['p1_ln0_g']), r(p['p1_ln0_b']), r(p['p1_ln1_g']), r(p['p1_ln1_b']),
             p['p2_q_W'], r(p['p2_q_b']), p['p2_k_W'], r(p['p2_k_b']),
             p['p2_v_W'], r(p['p2_v_b']),
             p['p2_o_W'], r(p['p2_o_b']),
             r(p['p2_ln0_g']), r(p['p2_ln0_b']), r(p['p2_ln1_g']), r(p['p2_ln1_b']),
             p['p3_S'].reshape(1, D),
             p['p3_q_W'], r(p['p3_q_b']), p['p3_k_W'], r(p['p3_k_b']),
             p['p3_v_W'], r(p['p3_v_b']),
             p['p3_o_W'], r(p['p3_o_b']),
             r(p['p3_ln0_g']), r(p['p3_ln0_b']), r(p['p3_ln1_g']), r(p['p3_ln1_b']),
             p['lin2_W'], r(p['lin2_b'])]

    y = pl.pallas_call(
        _readout_kernel,
        grid=(B,),
        in_specs=[_full(a.shape) for a in ro_in],
        out_specs=pl.BlockSpec((1, 1, DO), lambda i: (i, 0, 0)),
        out_shape=jax.ShapeDtypeStruct((B, 1, DO), jnp.float32),
    )(*ro_in)
    return y.reshape(B, DO)


# speedup vs baseline: 3.3987x; 3.3987x over previous
import jax
import jax.numpy as jnp
from jax.experimental import pallas as pl
from jax.experimental.pallas import tpu as pltpu

N = 10000
E = 320000
D = 128
DO = 64
B = 32
H = 8
HD = D // H
S1 = 75
EC = 1000
EG = E // EC
NEG = -1e9
SCALE = 1.0 / (128.0 ** 0.5)


def _scatter1_kernel(eb_ref, x_ref, agg_ref):
    @pl.when(pl.program_id(0) == 0)
    def _():
        agg_ref[...] = jnp.zeros_like(agg_ref)

    def body(j, carry):
        s = eb_ref[0, 0, j]
        d = eb_ref[0, 1, j]
        agg_ref[pl.ds(d, 1), :] = agg_ref[pl.ds(d, 1), :] + x_ref[pl.ds(s, 1), :]
        return carry

    jax.lax.fori_loop(0, EC, body, 0)


def _bn(h, g, b):
    m = jnp.mean(h, axis=0, keepdims=True)
    v = jnp.mean(jnp.square(h - m), axis=0, keepdims=True)
    return (h - m) * jax.lax.rsqrt(v + 1e-5) * g + b


def _ln(h, g, b):
    m = jnp.mean(h, axis=-1, keepdims=True)
    v = jnp.mean(jnp.square(h - m), axis=-1, keepdims=True)
    return (h - m) * jax.lax.rsqrt(v + 1e-5) * g + b


def _dense_kernel(x_ref, agg_ref,
                  w1_ref, b1_ref, g1_ref, bb1_ref,
                  w2_ref, b2_ref, g2_ref, bb2_ref,
                  lw_ref, lb_ref, kw_ref, vw_ref,
                  hws_ref, dinv_ref):
    h = x_ref[...] + agg_ref[:, 0:D]
    h = jnp.maximum(_bn(jnp.dot(h, w1_ref[...], preferred_element_type=jnp.float32)
                        + b1_ref[...], g1_ref[...], bb1_ref[...]), 0.0)
    h = jnp.maximum(_bn(jnp.dot(h, w2_ref[...], preferred_element_type=jnp.float32)
                        + b2_ref[...], g2_ref[...], bb2_ref[...]), 0.0)
    h1 = jnp.dot(h, lw_ref[...], preferred_element_type=jnp.float32) + lb_ref[...]
    dinv = jax.lax.rsqrt(agg_ref[:, D:D + 1] + 1.0)
    hwk = jnp.dot(h1, kw_ref[...], preferred_element_type=jnp.float32)
    hwv = jnp.dot(h1, vw_ref[...], preferred_element_type=jnp.float32)
    hws_ref[...] = jnp.concatenate([hwk, hwv], axis=1) * dinv
    dinv_ref[...] = dinv


def _scatter2_kernel(eb_ref, hws_ref, dinv_ref, kvb_ref, kv_ref):
    @pl.when(pl.program_id(0) == 0)
    def _():
        kv_ref[...] = hws_ref[...]

    def body(j, carry):
        s = eb_ref[0, 0, j]
        d = eb_ref[0, 1, j]
        kv_ref[pl.ds(d, 1), :] = kv_ref[pl.ds(d, 1), :] + hws_ref[pl.ds(s, 1), :]
        return carry

    jax.lax.fori_loop(0, EC, body, 0)

    @pl.when(pl.program_id(0) == EG - 1)
    def _():
        kv_ref[...] = kv_ref[...] * dinv_ref[...] + kvb_ref[...]


def _mab_heads(qp, k, v, mask_row):
    outs = []
    for h in range(H):
        qh = qp[:, h * HD:(h + 1) * HD]
        kh = k[:, h * HD:(h + 1) * HD]
        vh = v[:, h * HD:(h + 1) * HD]
        s = jax.lax.dot_general(qh, kh, (((1,), (1,)), ((), ())),
                                preferred_element_type=jnp.float32) * SCALE
        if mask_row is not None:
            s = s + mask_row
        m = jnp.max(s, axis=-1, keepdims=True)
        e = jnp.exp(s - m)
        a = e / jnp.sum(e, axis=-1, keepdims=True)
        outs.append(qh + jnp.dot(a, vh, preferred_element_type=jnp.float32))
    return jnp.concatenate(outs, axis=1)


def _mab_tail(o, ow, ob, g0, b0, g1, b1):
    o = _ln(o, g0, b0)
    o = o + jnp.maximum(jnp.dot(o, ow, preferred_element_type=jnp.float32) + ob, 0.0)
    return _ln(o, g1, b1)


def _readout_kernel(kn_ref, vn_ref, brow_ref, s1_ref,
                    q1w_ref, q1b_ref, o1w_ref, o1b_ref,
                    l10g_ref, l10b_ref, l11g_ref, l11b_ref,
                    q2w_ref, q2b_ref, k2w_ref, k2b_ref, v2w_ref, v2b_ref,
                    o2w_ref, o2b_ref, l20g_ref, l20b_ref, l21g_ref, l21b_ref,
                    s3_ref, q3w_ref, q3b_ref, k3w_ref, k3b_ref, v3w_ref, v3b_ref,
                    o3w_ref, o3b_ref, l30g_ref, l30b_ref, l31g_ref, l31b_ref,
                    l2w_ref, l2b_ref, out_ref):
    b = pl.program_id(0)
    mask_row = jnp.where(brow_ref[...] == b, 0.0, NEG)
    qp1 = jnp.dot(s1_ref[...], q1w_ref[...],
                  preferred_element_type=jnp.float32) + q1b_ref[...]
    o = _mab_heads(qp1, kn_ref[...], vn_ref[...], mask_row)
    out1 = _mab_tail(o, o1w_ref[...], o1b_ref[...], l10g_ref[...], l10b_ref[...],
                     l11g_ref[...], l11b_ref[...])
    qp2 = jnp.dot(out1, q2w_ref[...], preferred_element_type=jnp.float32) + q2b_ref[...]
    k2 = jnp.dot(out1, k2w_ref[...], preferred_element_type=jnp.float32) + k2b_ref[...]
    v2 = jnp.dot(out1, v2w_ref[...], preferred_element_type=jnp.float32) + v2b_ref[...]
    o = _mab_heads(qp2, k2, v2, None)
    out2 = _mab_tail(o, o2w_ref[...], o2b_ref[...], l20g_ref[...], l20b_ref[...],
                     l21g_ref[...], l21b_ref[...])
    qp3 = jnp.dot(s3_ref[...], q3w_ref[...],
                  preferred_element_type=jnp.float32) + q3b_ref[...]
    k3 = jnp.dot(out2, k3w_ref[...], preferred_element_type=jnp.float32) + k3b_ref[...]
    v3 = jnp.dot(out2, v3w_ref[...], preferred_element_type=jnp.float32) + v3b_ref[...]
    o = _mab_heads(qp3, k3, v3, None)
    out3 = _mab_tail(o, o3w_ref[...], o3b_ref[...], l30g_ref[...], l30b_ref[...],
                     l31g_ref[...], l31b_ref[...])
    y = jnp.dot(out3, l2w_ref[...],
                preferred_element_type=jnp.float32) + l2b_ref[...]
    out_ref[...] = y.reshape(1, 1, DO)


def _full(shape):
    return pl.BlockSpec(shape, lambda i: tuple(0 for _ in shape))


def kernel(x, edge_index, batch, params):
    p = params
    ei = edge_index.astype(jnp.int32).reshape(2, EG, EC).transpose(1, 0, 2)
    brow = batch.astype(jnp.int32).reshape(1, N)
    r = lambda a: a.reshape(1, -1)

    x_aug = jnp.concatenate(
        [x, jnp.ones((N, 1), jnp.float32), jnp.zeros((N, D - 1), jnp.float32)], axis=1)
    agg = pl.pallas_call(
        _scatter1_kernel,
        grid=(EG,),
        in_specs=[pl.BlockSpec((1, 2, EC), lambda i: (i, 0, 0), memory_space=pltpu.SMEM),
                  _full((N, 2 * D))],
        out_specs=_full((N, 2 * D)),
        out_shape=jax.ShapeDtypeStruct((N, 2 * D), jnp.float32),
    )(ei, x_aug)

    hws, dinv = pl.pallas_call(
        _dense_kernel,
        out_shape=[jax.ShapeDtypeStruct((N, 2 * D), jnp.float32),
                   jax.ShapeDtypeStruct((N, 1), jnp.float32)],
    )(x, agg,
      p['gin1_W'], r(p['gin1_b']), r(p['gin_bn1_g']), r(p['gin_bn1_b']),
      p['gin2_W'], r(p['gin2_b']), r(p['gin_bn2_g']), r(p['gin_bn2_b']),
      p['lin1_W'], r(p['lin1_b']), p['p1_k_W'], p['p1_v_W'])

    kvb = jnp.concatenate([p['p1_k_b'], p['p1_v_b']]).reshape(1, 2 * D)
    kv = pl.pallas_call(
        _scatter2_kernel,
        grid=(EG,),
        in_specs=[pl.BlockSpec((1, 2, EC), lambda i: (i, 0, 0), memory_space=pltpu.SMEM),
                  _full((N, 2 * D)), _full((N, 1)), _full((1, 2 * D))],
        out_specs=_full((N, 2 * D)),
        out_shape=jax.ShapeDtypeStruct((N, 2 * D), jnp.float32),
    )(ei, hws, dinv, kvb)
    kn = kv[:, :D]
    vn = kv[:, D:]

    ro_in = [kn, vn, brow, p['p1_S'].reshape(S1, D),
             p['p1_q_W'], r(p['p1_q_b']), p['p1_o_W'], r(p['p1_o_b']),
             r(p['p1_ln0_g']), r(p['p1_ln0_b']), r(p['p1_ln1_g']), r(p['p1_ln1_b']),
             p['p2_q_W'], r(p['p2_q_b']), p['p2_k_W'], r(p['p2_k_b']),
             p['p2_v_W'], r(p['p2_v_b']),
             p['p2_o_W'], r(p['p2_o_b']),
             r(p['p2_ln0_g']), r(p['p2_ln0_b']), r(p['p2_ln1_g']), r(p['p2_ln1_b']),
             p['p3_S'].reshape(1, D),
             p['p3_q_W'], r(p['p3_q_b']), p['p3_k_W'], r(p['p3_k_b']),
             p['p3_v_W'], r(p['p3_v_b']),
             p['p3_o_W'], r(p['p3_o_b']),
             r(p['p3_ln0_g']), r(p['p3_ln0_b']), r(p['p3_ln1_g']), r(p['p3_ln1_b']),
             p['lin2_W'], r(p['lin2_b'])]

    y = pl.pallas_call(
        _readout_kernel,
        grid=(B,),
        in_specs=[_full(a.shape) for a in ro_in],
        out_specs=pl.BlockSpec((1, 1, DO), lambda i: (i, 0, 0)),
        out_shape=jax.ShapeDtypeStruct((B, 1, DO), jnp.float32),
    )(*ro_in)
    return y.reshape(B, DO)


# speedup vs baseline: 5.2972x; 1.5586x over previous
import jax
import jax.numpy as jnp
from jax.experimental import pallas as pl
from jax.experimental.pallas import tpu as pltpu

N = 10000
E = 320000
D = 128
DO = 64
B = 32
H = 8
HD = D // H
S1 = 75
EC = 1000
EG = E // EC
NEG = -1e9
SCALE = 1.0 / (128.0 ** 0.5)


def _scatter1_kernel(eb_ref, x_ref, agg_ref):
    @pl.when(pl.program_id(0) == 0)
    def _():
        agg_ref[...] = jnp.zeros_like(agg_ref)

    def body(j, carry):
        for u in range(4):
            s = eb_ref[0, 0, 4 * j + u]
            d = eb_ref[0, 1, 4 * j + u]
            agg_ref[pl.ds(d, 1), :] = agg_ref[pl.ds(d, 1), :] + x_ref[pl.ds(s, 1), :]
        return carry

    jax.lax.fori_loop(0, EC // 4, body, 0)


def _bn(h, g, b):
    m = jnp.mean(h, axis=0, keepdims=True)
    v = jnp.mean(jnp.square(h - m), axis=0, keepdims=True)
    return (h - m) * jax.lax.rsqrt(v + 1e-5) * g + b


def _ln(h, g, b):
    m = jnp.mean(h, axis=-1, keepdims=True)
    v = jnp.mean(jnp.square(h - m), axis=-1, keepdims=True)
    return (h - m) * jax.lax.rsqrt(v + 1e-5) * g + b


def _dense_kernel(x_ref, agg_ref,
                  w1_ref, b1_ref, g1_ref, bb1_ref,
                  w2_ref, b2_ref, g2_ref, bb2_ref,
                  lw_ref, lb_ref, kw_ref, vw_ref,
                  hws_ref, dinv_ref):
    h = x_ref[...] + agg_ref[:, 0:D]
    h = jnp.maximum(_bn(jnp.dot(h, w1_ref[...], preferred_element_type=jnp.float32)
                        + b1_ref[...], g1_ref[...], bb1_ref[...]), 0.0)
    h = jnp.maximum(_bn(jnp.dot(h, w2_ref[...], preferred_element_type=jnp.float32)
                        + b2_ref[...], g2_ref[...], bb2_ref[...]), 0.0)
    h1 = jnp.dot(h, lw_ref[...], preferred_element_type=jnp.float32) + lb_ref[...]
    dinv = jax.lax.rsqrt(agg_ref[:, D:D + 1] + 1.0)
    hwk = jnp.dot(h1, kw_ref[...], preferred_element_type=jnp.float32)
    hwv = jnp.dot(h1, vw_ref[...], preferred_element_type=jnp.float32)
    hws_ref[...] = jnp.concatenate([hwk, hwv], axis=1) * dinv
    dinv_ref[...] = dinv


def _scatter2_kernel(eb_ref, hws_ref, dinv_ref, kvb_ref, kv_ref):
    @pl.when(pl.program_id(0) == 0)
    def _():
        kv_ref[...] = hws_ref[...]

    def body(j, carry):
        for u in range(4):
            s = eb_ref[0, 0, 4 * j + u]
            d = eb_ref[0, 1, 4 * j + u]
            kv_ref[pl.ds(d, 1), :] = kv_ref[pl.ds(d, 1), :] + hws_ref[pl.ds(s, 1), :]
        return carry

    jax.lax.fori_loop(0, EC // 4, body, 0)

    @pl.when(pl.program_id(0) == EG - 1)
    def _():
        kv_ref[...] = kv_ref[...] * dinv_ref[...] + kvb_ref[...]


def _mab_heads(qp, k, v, mask_row):
    outs = []
    for h in range(H):
        qh = qp[:, h * HD:(h + 1) * HD]
        kh = k[:, h * HD:(h + 1) * HD]
        vh = v[:, h * HD:(h + 1) * HD]
        s = jax.lax.dot_general(qh, kh, (((1,), (1,)), ((), ())),
                                preferred_element_type=jnp.float32) * SCALE
        if mask_row is not None:
            s = s + mask_row
        m = jnp.max(s, axis=-1, keepdims=True)
        e = jnp.exp(s - m)
        a = e / jnp.sum(e, axis=-1, keepdims=True)
        outs.append(qh + jnp.dot(a, vh, preferred_element_type=jnp.float32))
    return jnp.concatenate(outs, axis=1)


def _mab_tail(o, ow, ob, g0, b0, g1, b1):
    o = _ln(o, g0, b0)
    o = o + jnp.maximum(jnp.dot(o, ow, preferred_element_type=jnp.float32) + ob, 0.0)
    return _ln(o, g1, b1)


def _readout_kernel(kn_ref, vn_ref, brow_ref, s1_ref,
                    q1w_ref, q1b_ref, o1w_ref, o1b_ref,
                    l10g_ref, l10b_ref, l11g_ref, l11b_ref,
                    q2w_ref, q2b_ref, k2w_ref, k2b_ref, v2w_ref, v2b_ref,
                    o2w_ref, o2b_ref, l20g_ref, l20b_ref, l21g_ref, l21b_ref,
                    s3_ref, q3w_ref, q3b_ref, k3w_ref, k3b_ref, v3w_ref, v3b_ref,
                    o3w_ref, o3b_ref, l30g_ref, l30b_ref, l31g_ref, l31b_ref,
                    l2w_ref, l2b_ref, out_ref):
    b = pl.program_id(0)
    mask_row = jnp.where(brow_ref[...] == b, 0.0, NEG)
    qp1 = jnp.dot(s1_ref[...], q1w_ref[...],
                  preferred_element_type=jnp.float32) + q1b_ref[...]
    o = _mab_heads(qp1, kn_ref[...], vn_ref[...], mask_row)
    out1 = _mab_tail(o, o1w_ref[...], o1b_ref[...], l10g_ref[...], l10b_ref[...],
                     l11g_ref[...], l11b_ref[...])
    qp2 = jnp.dot(out1, q2w_ref[...], preferred_element_type=jnp.float32) + q2b_ref[...]
    k2 = jnp.dot(out1, k2w_ref[...], preferred_element_type=jnp.float32) + k2b_ref[...]
    v2 = jnp.dot(out1, v2w_ref[...], preferred_element_type=jnp.float32) + v2b_ref[...]
    o = _mab_heads(qp2, k2, v2, None)
    out2 = _mab_tail(o, o2w_ref[...], o2b_ref[...], l20g_ref[...], l20b_ref[...],
                     l21g_ref[...], l21b_ref[...])
    qp3 = jnp.dot(s3_ref[...], q3w_ref[...],
                  preferred_element_type=jnp.float32) + q3b_ref[...]
    k3 = jnp.dot(out2, k3w_ref[...], preferred_element_type=jnp.float32) + k3b_ref[...]
    v3 = jnp.dot(out2, v3w_ref[...], preferred_element_type=jnp.float32) + v3b_ref[...]
    o = _mab_heads(qp3, k3, v3, None)
    out3 = _mab_tail(o, o3w_ref[...], o3b_ref[...], l30g_ref[...], l30b_ref[...],
                     l31g_ref[...], l31b_ref[...])
    y = jnp.dot(out3, l2w_ref[...],
                preferred_element_type=jnp.float32) + l2b_ref[...]
    out_ref[...] = y.reshape(1, 1, DO)


def _full(shape):
    return pl.BlockSpec(shape, lambda i: tuple(0 for _ in shape))


def kernel(x, edge_index, batch, params):
    p = params
    ei = edge_index.astype(jnp.int32).reshape(2, EG, EC).transpose(1, 0, 2)
    brow = batch.astype(jnp.int32).reshape(1, N)
    r = lambda a: a.reshape(1, -1)

    x_aug = jnp.concatenate(
        [x, jnp.ones((N, 1), jnp.float32), jnp.zeros((N, D - 1), jnp.float32)], axis=1)
    agg = pl.pallas_call(
        _scatter1_kernel,
        grid=(EG,),
        in_specs=[pl.BlockSpec((1, 2, EC), lambda i: (i, 0, 0), memory_space=pltpu.SMEM),
                  _full((N, 2 * D))],
        out_specs=_full((N, 2 * D)),
        out_shape=jax.ShapeDtypeStruct((N, 2 * D), jnp.float32),
    )(ei, x_aug)

    hws, dinv = pl.pallas_call(
        _dense_kernel,
        out_shape=[jax.ShapeDtypeStruct((N, 2 * D), jnp.float32),
                   jax.ShapeDtypeStruct((N, 1), jnp.float32)],
    )(x, agg,
      p['gin1_W'], r(p['gin1_b']), r(p['gin_bn1_g']), r(p['gin_bn1_b']),
      p['gin2_W'], r(p['gin2_b']), r(p['gin_bn2_g']), r(p['gin_bn2_b']),
      p['lin1_W'], r(p['lin1_b']), p['p1_k_W'], p['p1_v_W'])

    kvb = jnp.concatenate([p['p1_k_b'], p['p1_v_b']]).reshape(1, 2 * D)
    kv = pl.pallas_call(
        _scatter2_kernel,
        grid=(EG,),
        in_specs=[pl.BlockSpec((1, 2, EC), lambda i: (i, 0, 0), memory_space=pltpu.SMEM),
                  _full((N, 2 * D)), _full((N, 1)), _full((1, 2 * D))],
        out_specs=_full((N, 2 * D)),
        out_shape=jax.ShapeDtypeStruct((N, 2 * D), jnp.float32),
    )(ei, hws, dinv, kvb)
    kn = kv[:, :D]
    vn = kv[:, D:]

    ro_in = [kn, vn, brow, p['p1_S'].reshape(S1, D),
             p['p1_q_W'], r(p['p1_q_b']), p['p1_o_W'], r(p['p1_o_b']),
             r(p['p1_ln0_g']), r(p['p1_ln0_b']), r(p['p1_ln1_g']), r(p['p1_ln1_b']),
             p['p2_q_W'], r(p['p2_q_b']), p['p2_k_W'], r(p['p2_k_b']),
             p['p2_v_W'], r(p['p2_v_b']),
             p['p2_o_W'], r(p['p2_o_b']),
             r(p['p2_ln0_g']), r(p['p2_ln0_b']), r(p['p2_ln1_g']), r(p['p2_ln1_b']),
             p['p3_S'].reshape(1, D),
             p['p3_q_W'], r(p['p3_q_b']), p['p3_k_W'], r(p['p3_k_b']),
             p['p3_v_W'], r(p['p3_v_b']),
             p['p3_o_W'], r(p['p3_o_b']),
             r(p['p3_ln0_g']), r(p['p3_ln0_b']), r(p['p3_ln1_g']), r(p['p3_ln1_b']),
             p['lin2_W'], r(p['lin2_b'])]

    y = pl.pallas_call(
        _readout_kernel,
        grid=(B,),
        in_specs=[_full(a.shape) for a in ro_in],
        out_specs=pl.BlockSpec((1, 1, DO), lambda i: (i, 0, 0)),
        out_shape=jax.ShapeDtypeStruct((B, 1, DO), jnp.float32),
    )(*ro_in)
    return y.reshape(B, DO)
